# speedup vs baseline: 9.0484x; 9.0484x over previous
import math

import jax
import jax.numpy as jnp
from jax.experimental import pallas as pl
from jax.experimental.pallas import tpu as pltpu

S = 2048
D = 1024
KC = 5
F_BR = 4
BD = 128
LB = 128
TK = 8
PC = 64
NEG = -1e30
BLK = 256
NBLK = S // BLK
WIN = BLK + LB
SCALE = 1.0 / math.sqrt(D)


def _mm_t(x, w):
    return jax.lax.dot_general(x, w, (((1,), (1,)), ((), ())),
                               preferred_element_type=jnp.float32)


def _body(h_ref, Wct_ref, bconv_ref, Wq_ref, bq_ref, Wk_ref, bk_ref,
          Wv_ref, bv_ref, Wpk_ref, bpk_ref, Wpv_ref, bpv_ref,
          Wprop_ref, bprop_ref, Wback_ref, bback_ref,
          ws1_ref, wg1_ref, wg2_ref, u_ref, csc_ref, bgate_ref,
          out_ref, ktail, vtail, pklast, pvlast, htail):
    i = pl.program_id(0)
    start = i * BLK
    bf = jnp.bfloat16

    h_blk = h_ref[...]
    halo = jnp.where(i == 0, jnp.zeros_like(htail[...]), htail[...])
    hx = jnp.concatenate([halo, h_blk], axis=0)
    acc = jnp.broadcast_to(bconv_ref[...], (BLK, D))
    for j in range(KC):
        acc = acc + hx[j:j + BLK, :] * Wct_ref[j:j + 1, :]
    h1 = h_blk + acc
    htail[...] = h_blk[BLK - (KC - 1):, :]

    r_i = jax.lax.broadcasted_iota(jnp.int32, (BLK, BLK), 0)
    c_i = jax.lax.broadcasted_iota(jnp.int32, (BLK, BLK), 1)
    same_chunk = (r_i // PC) == (c_i // PC)
    denom = (r_i % PC + 1).astype(jnp.float32)
    A = jnp.where(same_chunk & (c_i <= r_i), 1.0 / denom, 0.0)
    ph = jnp.dot(A, h1, preferred_element_type=jnp.float32)

    h1b = h1.astype(bf)
    phb = ph.astype(bf)
    q = _mm_t(h1b, Wq_ref[...]) + bq_ref[...]
    k = _mm_t(h1b, Wk_ref[...]) + bk_ref[...]
    v = _mm_t(h1b, Wv_ref[...]) + bv_ref[...]
    pk = _mm_t(phb, Wpk_ref[...]) + bpk_ref[...]
    pv = _mm_t(phb, Wpv_ref[...]) + bpv_ref[...]
    p = _mm_t(h1b, Wprop_ref[...]) + bprop_ref[...]

    kb = k.astype(bf)
    vb = v.astype(bf)
    kt = jnp.where(i == 0, jnp.zeros_like(ktail[...]), ktail[...])
    vt = jnp.where(i == 0, jnp.zeros_like(vtail[...]), vtail[...])
    k_win = jnp.concatenate([kt, kb], axis=0)
    v_win = jnp.concatenate([vt, vb], axis=0)
    qb = q.astype(bf)
    scores = _mm_t(qb, k_win) * SCALE

    tmat = start + jax.lax.broadcasted_iota(jnp.int32, (BLK, WIN), 0)
    pos = (start - LB) + jax.lax.broadcasted_iota(jnp.int32, (BLK, WIN), 1)
    validm = (pos >= tmat - LB) & (pos <= tmat - 1) & (pos >= 0)
    scores = jnp.where(validm, scores, NEG)

    seq_s = jnp.sum(jnp.where(pos == tmat - 1, scores, 0.0), axis=1,
                    keepdims=True)

    pk_prev = jnp.concatenate(
        [jnp.where(i == 0, pk[0:1, :], pklast[...]), pk[:BLK - 1, :]], axis=0)
    pv_prev = jnp.concatenate(
        [jnp.where(i == 0, pv[0:1, :], pvlast[...]), pv[:BLK - 1, :]], axis=0)
    v_last = jnp.where(i == 0, v[0:1, :],
                       vtail[LB - 1:LB, :].astype(jnp.float32))
    v_prev = jnp.concatenate([v_last, v[:BLK - 1, :]], axis=0)

    ph_s = jnp.sum(q * pk_prev, axis=1, keepdims=True) * SCALE

    colid = jax.lax.broadcasted_iota(jnp.int32, (BLK, WIN), 1)
    m0 = jnp.max(scores, axis=1, keepdims=True)
    M = jnp.maximum(jnp.maximum(m0, seq_s), ph_s)
    e_seq = jnp.exp(seq_s - M)
    e_ph = jnp.exp(ph_s - M)
    den = e_seq + e_ph
    wnum = jnp.zeros((BLK, WIN), jnp.float32)
    cur = scores
    minf = jnp.float32(-jnp.inf)
    for it in range(TK):
        m = m0 if it == 0 else jnp.max(cur, axis=1, keepdims=True)
        sel = jnp.min(jnp.where(cur == m, colid, WIN), axis=1, keepdims=True)
        hot = colid == sel
        e = jnp.exp(m - M)
        den = den + e
        wnum = wnum + jnp.where(hot, e, 0.0)
        if it < TK - 1:
            cur = jnp.where(hot, minf, cur)

    sem_part = jax.lax.dot_general(wnum.astype(bf), v_win,
                                   (((1,), (0,)), ((), ())),
                                   preferred_element_type=jnp.float32)
    summ = (sem_part + e_seq * v_prev + e_ph * pv_prev) / den
    trow = start + jax.lax.broadcasted_iota(jnp.int32, (BLK, 1), 0)
    summ = jnp.where(trow >= 1, summ, 0.0)

    h1s = jnp.sum(h1 * ws1_ref[...], axis=1, keepdims=True)
    u = u_ref[...]
    pfs = [p[:, f * BD:(f + 1) * BD] for f in range(F_BR)]
    scs = [h1s + jnp.sum(pf * u, axis=1, keepdims=True) + csc_ref[...]
           for pf in pfs]
    mx = jnp.maximum(jnp.maximum(scs[0], scs[1]), jnp.maximum(scs[2], scs[3]))
    es = [jnp.exp(s - mx) for s in scs]
    sume = es[0] + es[1] + es[2] + es[3]
    pmix = (es[0] * pfs[0] + es[1] * pfs[1]
            + es[2] * pfs[2] + es[3] * pfs[3]) / sume
    branch = _mm_t(pmix.astype(bf), Wback_ref[...]) + bback_ref[...]
    bs = branch + summ

    gl = (jnp.sum(h1 * wg1_ref[...], axis=1, keepdims=True)
          + jnp.sum(bs * wg2_ref[...], axis=1, keepdims=True)
          + bgate_ref[...])
    g = jax.nn.sigmoid(gl)
    out_ref[...] = h1 + g * (bs - h1)

    ktail[...] = kb[BLK - LB:, :]
    vtail[...] = vb[BLK - LB:, :]
    pklast[...] = pk[BLK - 1:, :]
    pvlast[...] = pv[BLK - 1:, :]


def kernel(h, Wconv, bconv, Wq, bq, Wk, bk, Wv, bv, Wpk, bpk, Wpv, bpv,
           Wprop, bprop, Wback, bback, Wscore, bscore, Wgate, bgate):
    bf = jnp.bfloat16
    h2 = h.reshape(S, D)
    ws1 = Wscore[:D].reshape(1, D)
    ws2 = Wscore[D:].reshape(1, D)
    wg1 = Wgate[:D].reshape(1, D)
    wg2 = Wgate[D:].reshape(1, D)
    u = jnp.dot(ws2, Wback).reshape(1, BD)
    csc = (jnp.sum(bback * ws2) + bscore).reshape(1, 1)
    bg = bgate.reshape(1, 1)

    full = lambda s: pl.BlockSpec(s, lambda i: (0, 0))
    blocked = pl.BlockSpec((BLK, D), lambda i: (i, 0))

    out = pl.pallas_call(
        _body,
        grid=(NBLK,),
        in_specs=[
            blocked,
            full((KC, D)),
            full((1, D)),
            full((D, D)), full((1, D)),
            full((D, D)), full((1, D)),
            full((D, D)), full((1, D)),
            full((D, D)), full((1, D)),
            full((D, D)), full((1, D)),
            full((F_BR * BD, D)), full((1, F_BR * BD)),
            full((D, BD)), full((1, D)),
            full((1, D)),
            full((1, D)), full((1, D)),
            full((1, BD)),
            full((1, 1)), full((1, 1)),
        ],
        out_specs=blocked,
        out_shape=jax.ShapeDtypeStruct((S, D), jnp.float32),
        scratch_shapes=[
            pltpu.VMEM((LB, D), bf),
            pltpu.VMEM((LB, D), bf),
            pltpu.VMEM((1, D), jnp.float32),
            pltpu.VMEM((1, D), jnp.float32),
            pltpu.VMEM((KC - 1, D), jnp.float32),
        ],
    )(
        h2, Wconv.T, bconv.reshape(1, D),
        Wq.astype(bf), bq.reshape(1, D),
        Wk.astype(bf), bk.reshape(1, D),
        Wv.astype(bf), bv.reshape(1, D),
        Wpk.astype(bf), bpk.reshape(1, D),
        Wpv.astype(bf), bpv.reshape(1, D),
        Wprop.astype(bf), bprop.reshape(1, F_BR * BD),
        Wback.astype(bf), bback.reshape(1, D),
        ws1, wg1, wg2, u, csc, bg,
    )
    return out.reshape(1, S, D)
